# baseline (speedup 1.0000x reference)
import functools

import jax
import jax.numpy as jnp
from jax.experimental import pallas as pl
from jax.experimental.pallas import tpu as pltpu

_F32 = jnp.float32


def _se_fused_kernel(x_ref, w1_ref, b1_ref, w2_ref, b2_ref, o_ref, *, inv_hw):
    xs = x_ref[0].astype(_F32)
    srow = jnp.sum(xs, axis=2, keepdims=True)
    mean = jnp.sum(srow, axis=1) * inv_hw
    h = jnp.dot(w1_ref[...], mean, preferred_element_type=_F32)
    h = jnp.maximum(h + b1_ref[...], 0.0)
    s = jnp.dot(w2_ref[...], h, preferred_element_type=_F32)
    scale = jax.nn.sigmoid(s + b2_ref[...])
    o_ref[0] = (xs * scale[:, :, None]).astype(o_ref.dtype)


def kernel(x_nchw, w1, b1, w2, b2):
    N, C, H, W = x_nchw.shape
    HW = H * W
    mid = w1.shape[0]

    w1m = w1.reshape(mid, C).astype(_F32)
    b1m = b1.reshape(mid, 1).astype(_F32)
    w2m = w2.reshape(C, mid).astype(_F32)
    b2m = b2.reshape(C, 1).astype(_F32)

    return pl.pallas_call(
        functools.partial(_se_fused_kernel, inv_hw=1.0 / HW),
        out_shape=jax.ShapeDtypeStruct((N, C, H, W), x_nchw.dtype),
        grid=(N,),
        in_specs=[
            pl.BlockSpec((1, C, H, W), lambda n: (n, 0, 0, 0)),
            pl.BlockSpec((mid, C), lambda n: (0, 0)),
            pl.BlockSpec((mid, 1), lambda n: (0, 0)),
            pl.BlockSpec((C, mid), lambda n: (0, 0)),
            pl.BlockSpec((C, 1), lambda n: (0, 0)),
        ],
        out_specs=pl.BlockSpec((1, C, H, W), lambda n: (n, 0, 0, 0)),
        compiler_params=pltpu.CompilerParams(
            dimension_semantics=("parallel",)),
    )(x_nchw, w1m, b1m, w2m, b2m)


# speedup vs baseline: 10.3322x; 10.3322x over previous
import functools

import jax
import jax.numpy as jnp
from jax.experimental import pallas as pl
from jax.experimental.pallas import tpu as pltpu

_F32 = jnp.float32


def _se_fused_kernel(x_ref, w1_ref, b1_ref, w2_ref, b2_ref, o_ref, *, inv_hw):
    xs = x_ref[...].astype(_F32)
    mean = jnp.sum(xs, axis=0) * inv_hw
    h = jnp.dot(mean, w1_ref[...], preferred_element_type=_F32)
    h = jnp.maximum(h + b1_ref[...], 0.0)
    s = jnp.dot(h, w2_ref[...], preferred_element_type=_F32)
    scale = jax.nn.sigmoid(s + b2_ref[...])
    o_ref[...] = (xs * scale[None, :, :]).astype(o_ref.dtype)


def kernel(x_nchw, w1, b1, w2, b2):
    N, C, H, W = x_nchw.shape
    HW = H * W
    mid = w1.shape[0]

    x_t = jnp.transpose(x_nchw, (2, 3, 0, 1)).reshape(HW, N, C)

    w1m = w1.reshape(mid, C).T.astype(_F32)
    b1m = b1.reshape(1, mid).astype(_F32)
    w2m = w2.reshape(C, mid).T.astype(_F32)
    b2m = b2.reshape(1, C).astype(_F32)

    NT = 8 if N % 8 == 0 else N
    out_t = pl.pallas_call(
        functools.partial(_se_fused_kernel, inv_hw=1.0 / HW),
        out_shape=jax.ShapeDtypeStruct((HW, N, C), x_nchw.dtype),
        grid=(N // NT,),
        in_specs=[
            pl.BlockSpec((HW, NT, C), lambda n: (0, n, 0)),
            pl.BlockSpec((C, mid), lambda n: (0, 0)),
            pl.BlockSpec((1, mid), lambda n: (0, 0)),
            pl.BlockSpec((mid, C), lambda n: (0, 0)),
            pl.BlockSpec((1, C), lambda n: (0, 0)),
        ],
        out_specs=pl.BlockSpec((HW, NT, C), lambda n: (0, n, 0)),
        compiler_params=pltpu.CompilerParams(
            dimension_semantics=("parallel",)),
    )(x_t, w1m, b1m, w2m, b2m)

    return jnp.transpose(out_t.reshape(H, W, N, C), (2, 3, 0, 1))


# speedup vs baseline: 11.8505x; 1.1470x over previous
import functools

import jax
import jax.numpy as jnp
from jax.experimental import pallas as pl
from jax.experimental.pallas import tpu as pltpu

_F32 = jnp.float32


def _se_fused_kernel(x_ref, w1_ref, b1_ref, w2_ref, b2_ref, o_ref, *, inv_hw):
    xs = x_ref[...].astype(_F32)
    mean = jnp.sum(xs, axis=0) * inv_hw
    h = jnp.dot(mean, w1_ref[...], preferred_element_type=_F32)
    h = jnp.maximum(h + b1_ref[...], 0.0)
    s = jnp.dot(h, w2_ref[...], preferred_element_type=_F32)
    scale = jax.nn.sigmoid(s + b2_ref[...])
    o_ref[...] = (xs * scale[None, :, :]).astype(o_ref.dtype)


def kernel(x_nchw, w1, b1, w2, b2):
    N, C, H, W = x_nchw.shape
    HW = H * W
    mid = w1.shape[0]

    x_t = jnp.transpose(x_nchw, (2, 3, 0, 1)).reshape(HW, N, C)

    w1m = w1.reshape(mid, C).T.astype(_F32)
    b1m = b1.reshape(1, mid).astype(_F32)
    w2m = w2.reshape(C, mid).T.astype(_F32)
    b2m = b2.reshape(1, C).astype(_F32)

    NT = 16 if N % 16 == 0 else N
    out_t = pl.pallas_call(
        functools.partial(_se_fused_kernel, inv_hw=1.0 / HW),
        out_shape=jax.ShapeDtypeStruct((HW, N, C), x_nchw.dtype),
        grid=(N // NT,),
        in_specs=[
            pl.BlockSpec((HW, NT, C), lambda n: (0, n, 0)),
            pl.BlockSpec((C, mid), lambda n: (0, 0)),
            pl.BlockSpec((1, mid), lambda n: (0, 0)),
            pl.BlockSpec((mid, C), lambda n: (0, 0)),
            pl.BlockSpec((1, C), lambda n: (0, 0)),
        ],
        out_specs=pl.BlockSpec((HW, NT, C), lambda n: (0, n, 0)),
        compiler_params=pltpu.CompilerParams(
            dimension_semantics=("parallel",)),
    )(x_t, w1m, b1m, w2m, b2m)

    return jnp.transpose(out_t.reshape(H, W, N, C), (2, 3, 0, 1))
